# baseline (speedup 1.0000x reference)
import jax
import jax.numpy as jnp
from jax.experimental import pallas as pl

HEADS = 4
HD = [64, 213, 362, 512]
NUM_GRAPHS = 64


def _dense(x, W, b, gamma=None, beta=None):
    y = jnp.dot(x, W) + b
    y = y * jax.nn.sigmoid(y)
    if gamma is not None:
        mu = jnp.mean(y, axis=-1, keepdims=True)
        var = jnp.var(y, axis=-1, keepdims=True)
        y = (y - mu) / jnp.sqrt(var + 1e-5) * gamma + beta
    return y


def _gatv2(x, ei, ea, p, num_nodes, out_ch, emask):
    src = ei[0]
    dst = ei[1]
    xl = (jnp.dot(x, p['Wl']) + p['bl']).reshape(-1, HEADS, out_ch)
    xr = (jnp.dot(x, p['Wr']) + p['br']).reshape(-1, HEADS, out_ch)
    e = jnp.dot(ea, p['We']).reshape(-1, HEADS, out_ch)
    m = xl[src] + xr[dst] + e
    m = jax.nn.leaky_relu(m, 0.2)
    logits = jnp.sum(m * p['att'][None], axis=-1)
    masked_logits = jnp.where(emask[:, None], logits, -jnp.inf)
    lmax = jax.ops.segment_max(masked_logits, dst, num_segments=num_nodes)
    lmax = jnp.where(jnp.isfinite(lmax), lmax, 0.0)
    ex = jnp.where(emask[:, None], jnp.exp(logits - jax.lax.stop_gradient(lmax)[dst]), 0.0)
    den = jax.ops.segment_sum(ex, dst, num_segments=num_nodes)
    alpha = ex / (den[dst] + 1e-16)
    out = jax.ops.segment_sum(xl[src] * alpha[..., None], dst, num_segments=num_nodes)
    return jnp.mean(out, axis=1) + p['bias']


def _graphconv_score(x, ei, p, num_nodes, emask):
    agg = jax.ops.segment_sum(jnp.where(emask[:, None], x[ei[0]], 0.0), ei[1], num_segments=num_nodes)
    out = jnp.dot(agg, p['W_rel']) + p['b_rel'] + jnp.dot(x, p['W_root'])
    return out[:, 0]


def _mean_pool(x, batch, num_graphs):
    s = jax.ops.segment_sum(x, batch, num_segments=num_graphs + 1)[:num_graphs]
    c = jax.ops.segment_sum(jnp.ones((x.shape[0],), x.dtype), batch, num_segments=num_graphs + 1)[:num_graphs]
    return s / jnp.maximum(c, 1.0)[:, None]


def _build_pool_plan(score, ei, batch, emask, num_graphs, ratio=0.5):
    N = score.shape[0]
    order = jnp.lexsort((-score, batch))
    batch_sorted = batch[order]
    counts = jax.ops.segment_sum(jnp.ones_like(batch), batch, num_segments=num_graphs + 1)
    k = jnp.ceil(ratio * counts).astype(counts.dtype)
    starts = jnp.concatenate([jnp.zeros((1,), counts.dtype), jnp.cumsum(counts)[:-1]])
    rank = jnp.arange(N, dtype=counts.dtype) - starts[batch_sorted]
    keep_sorted = (rank < k[batch_sorted]) & (batch_sorted < num_graphs)
    order2 = jnp.argsort(~keep_sorted)
    perm = order[order2]
    nkeep = jnp.sum(keep_sorted.astype(counts.dtype))
    alive_new = jnp.arange(N, dtype=counts.dtype) < nkeep
    new_batch = jnp.where(alive_new, batch[perm], num_graphs)
    remap = jnp.zeros((N,), ei.dtype).at[perm].set(jnp.arange(N, dtype=ei.dtype))
    s0 = remap[ei[0]]
    d0 = remap[ei[1]]
    new_emask = emask & (s0 < nkeep) & (d0 < nkeep)
    new_ei = jnp.where(new_emask[None, :], jnp.stack([s0, d0]), 0)
    return perm, new_ei, new_emask, new_batch


def kernel(x, edge_index, edge_attr, batch, params):
    num_graphs = NUM_GRAPHS
    h = _dense(x, params['node_W'], params['node_b'])
    ea = _dense(edge_attr, params['edge_W'], params['edge_b'])
    ei = edge_index
    bt = batch
    emask = jnp.ones((ei.shape[1],), bool)
    graph_out = 0.0
    for i in range(3):
        N = h.shape[0]
        h = _gatv2(h, ei, ea, params['gat%d' % i], N, HD[i + 1], emask)
        pooled = _mean_pool(h, bt, num_graphs)
        graph_out = graph_out + _dense(pooled, params['gm%d_W' % i], params['gm%d_b' % i], params['gm%d_g' % i], params['gm%d_be' % i])
        if i < 2:
            h = jax.nn.leaky_relu(h, 0.01)
            score = jnp.tanh(_graphconv_score(h, ei, params['pool%d' % i], N, emask))
            plan = _build_pool_plan(score, ei, bt, emask, num_graphs)
            perm, new_ei, new_emask, new_bt = plan
            h = h[perm] * score[perm][:, None]
            ei = new_ei
            emask = new_emask
            bt = new_bt
    return _dense(graph_out, params['fc_W'], params['fc_b'])


# speedup vs baseline: 1.6194x; 1.6194x over previous
import functools

import jax
import jax.numpy as jnp
from jax import lax
from jax.experimental import pallas as pl
from jax.experimental.pallas import tpu as pltpu
from jax.experimental.pallas import tpu_sc as plsc

HEADS = 4
HD = [64, 213, 362, 512]
HDP = [64, 256, 384, 512]
NUM_GRAPHS = 64

_SC_CORES = 2
_SC_SUBCORES = 16
_SC_WORKERS = _SC_CORES * _SC_SUBCORES
_SC_CHUNK = 40


def sc_gather(table, idx):
    V, D = table.shape
    E = idx.shape[0]
    assert D % 16 == 0
    step = _SC_WORKERS * _SC_CHUNK
    Ep = ((E + step - 1) // step) * step
    if Ep != E:
        idx = jnp.pad(idx, (0, Ep - E))
    per_w = Ep // _SC_WORKERS
    n_iters = per_w // _SC_CHUNK
    K = _SC_CHUNK
    mesh = plsc.VectorSubcoreMesh(core_axis_name="c", subcore_axis_name="s")

    @functools.partial(
        pl.kernel,
        mesh=mesh,
        out_type=jax.ShapeDtypeStruct((Ep, D), jnp.float32),
        scratch_types=[
            pltpu.VMEM((K,), jnp.int32),
            pltpu.VMEM((K, D), jnp.float32),
            pltpu.SemaphoreType.DMA,
        ],
    )
    def k(table_hbm, idx_hbm, out_hbm, idx_v, rows_v, sem):
        wid = lax.axis_index("s") * _SC_CORES + lax.axis_index("c")
        base = wid * per_w

        def body(i, carry):
            off = base + i * K
            pltpu.sync_copy(idx_hbm.at[pl.ds(off, K)], idx_v)
            pltpu.async_copy(table_hbm.at[idx_v], rows_v, sem).wait()
            pltpu.sync_copy(rows_v, out_hbm.at[pl.ds(off, K)])
            return carry

        lax.fori_loop(0, n_iters, body, 0)

    out = k(table, idx)
    return out[:E] if Ep != E else out


def _dense(x, W, b, gamma=None, beta=None, silu=True):
    M, Kd = x.shape
    Nf = W.shape[1]
    BM = 256
    gm = pl.cdiv(M, BM)
    ln = gamma is not None
    if b is None:
        b = jnp.zeros((Nf,), jnp.float32)

    def body(x_ref, w_ref, b_ref, *rest):
        if ln:
            g_ref, be_ref, o_ref = rest
        else:
            (o_ref,) = rest
        y = jnp.dot(x_ref[...], w_ref[...], preferred_element_type=jnp.float32, precision=jax.lax.Precision.HIGHEST)
        y = y + b_ref[...]
        if silu:
            y = y * jax.nn.sigmoid(y)
        if ln:
            mu = jnp.mean(y, axis=-1, keepdims=True)
            var = jnp.mean((y - mu) * (y - mu), axis=-1, keepdims=True)
            y = (y - mu) * jax.lax.rsqrt(var + 1e-5) * g_ref[...] + be_ref[...]
        o_ref[...] = y

    ins = [
        pl.BlockSpec((BM, Kd), lambda i: (i, 0)),
        pl.BlockSpec((Kd, Nf), lambda i: (0, 0)),
        pl.BlockSpec((1, Nf), lambda i: (0, 0)),
    ]
    args = [x, W, b.reshape(1, Nf)]
    if ln:
        ins += [pl.BlockSpec((1, Nf), lambda i: (0, 0))] * 2
        args += [gamma.reshape(1, Nf), beta.reshape(1, Nf)]
    return pl.pallas_call(
        body,
        grid=(gm,),
        in_specs=ins,
        out_specs=pl.BlockSpec((BM, Nf), lambda i: (i, 0)),
        out_shape=jax.ShapeDtypeStruct((M, Nf), jnp.float32),
    )(*args)


def _edge_logits(xls, xrs, ep, att_flat, fp):
    E, HF = xls.shape
    BE = 320
    ge = E // BE
    sel = (jnp.arange(HF, dtype=jnp.int32)[:, None] // fp
           == jnp.arange(128, dtype=jnp.int32)[None, :]).astype(jnp.float32)

    def body(a_ref, b_ref, c_ref, att_ref, sel_ref, o_ref):
        m = a_ref[...] + b_ref[...] + c_ref[...]
        m = jnp.where(m >= 0, m, 0.2 * m) * att_ref[...]
        o_ref[...] = jnp.dot(m, sel_ref[...], preferred_element_type=jnp.float32, precision=jax.lax.Precision.HIGHEST)

    out = pl.pallas_call(
        body,
        grid=(ge,),
        in_specs=[
            pl.BlockSpec((BE, HF), lambda i: (i, 0)),
            pl.BlockSpec((BE, HF), lambda i: (i, 0)),
            pl.BlockSpec((BE, HF), lambda i: (i, 0)),
            pl.BlockSpec((1, HF), lambda i: (0, 0)),
            pl.BlockSpec((HF, 128), lambda i: (0, 0)),
        ],
        out_specs=pl.BlockSpec((BE, 128), lambda i: (i, 0)),
        out_shape=jax.ShapeDtypeStruct((E, 128), jnp.float32),
    )(xls, xrs, ep, att_flat.reshape(1, HF), sel)
    return out[:, :HEADS]


def _aggregate(rows, wT, dst_s, num_nodes, n_heads, fp, scale, bias):
    E, HF = rows.shape
    EB = 640
    NB = 256
    n_eb = E // EB
    n_nb = pl.cdiv(num_nodes, NB)
    npad = n_nb * NB

    nb_edges = jnp.searchsorted(
        dst_s, jnp.arange(n_nb + 1, dtype=jnp.int32) * NB).astype(jnp.int32)
    lo = jnp.minimum(nb_edges[:-1] // EB, n_eb - 1)
    hi = (jnp.maximum(nb_edges[1:], 1) - 1) // EB
    cnt = jnp.maximum(hi - lo + 1, 1)
    S = n_eb + 2 * n_nb
    off = jnp.cumsum(cnt)
    pos = jnp.arange(S, dtype=jnp.int32)
    i_of = jnp.searchsorted(off, pos, side="right").astype(jnp.int32)
    i_cl = jnp.minimum(i_of, n_nb - 1)
    within = pos - jnp.where(i_cl > 0, off[jnp.maximum(i_cl - 1, 0)], 0)
    j_of = jnp.clip(lo[i_cl] + within, 0, n_eb - 1).astype(jnp.int32)
    valid = ((i_of < n_nb) & (within < (hi - lo + 1)[i_cl])).astype(jnp.int32)
    prev_i = jnp.concatenate([jnp.array([-1], jnp.int32), i_cl[:-1]])
    next_i = jnp.concatenate([i_cl[1:], jnp.array([-7], jnp.int32)])
    first = (i_cl != prev_i).astype(jnp.int32)
    last = (i_cl != next_i).astype(jnp.int32)

    dst3 = dst_s.reshape(n_eb, 1, EB)

    def body(nb_r, ej_r, val_r, first_r, last_r, dst_r, w_r, rows_r, b_ref,
             o_ref, acc):
        s = pl.program_id(0)

        @pl.when(first_r[s] == 1)
        def _():
            acc[...] = jnp.zeros_like(acc)

        node0 = nb_r[s] * NB
        rid = node0 + lax.broadcasted_iota(jnp.int32, (NB, EB), 0)
        hit = ((rid == dst_r[0]).astype(jnp.float32)
               * val_r[s].astype(jnp.float32))
        for h in range(n_heads):
            oh = hit * w_r[h:h + 1, :]
            acc[:, h * fp:(h + 1) * fp] += jnp.dot(
                oh, rows_r[:, h * fp:(h + 1) * fp],
                preferred_element_type=jnp.float32, precision=jax.lax.Precision.HIGHEST)

        @pl.when(last_r[s] == 1)
        def _():
            tot = acc[:, 0:fp]
            for h in range(1, n_heads):
                tot = tot + acc[:, h * fp:(h + 1) * fp]
            o_ref[...] = tot * scale + b_ref[...]

    grid_spec = pltpu.PrefetchScalarGridSpec(
        num_scalar_prefetch=5,
        grid=(S,),
        in_specs=[
            pl.BlockSpec((1, 1, EB), lambda s, nb, ej, va, fi, la: (ej[s], 0, 0)),
            pl.BlockSpec((8, EB), lambda s, nb, ej, va, fi, la: (0, ej[s])),
            pl.BlockSpec((EB, HF), lambda s, nb, ej, va, fi, la: (ej[s], 0)),
            pl.BlockSpec((1, fp), lambda s, nb, ej, va, fi, la: (0, 0)),
        ],
        out_specs=pl.BlockSpec((NB, fp), lambda s, nb, ej, va, fi, la: (nb[s], 0)),
        scratch_shapes=[pltpu.VMEM((NB, HF), jnp.float32)],
    )
    out = pl.pallas_call(
        body,
        grid_spec=grid_spec,
        out_shape=jax.ShapeDtypeStruct((npad, fp), jnp.float32),
    )(i_cl, j_of, valid, first, last, dst3, wT, rows, bias.reshape(1, fp))
    return out[:num_nodes]


def _sum_pool(h, batch, num_graphs):
    N, F = h.shape
    BN = 256
    g = pl.cdiv(N, BN)
    npad = g * BN
    b3 = jnp.full((npad,), num_graphs + 1, jnp.int32)
    b3 = b3.at[:N].set(batch.astype(jnp.int32)).reshape(g, 1, BN)

    def body(b_ref, h_ref, o_ref):
        s = pl.program_id(0)

        @pl.when(s == 0)
        def _():
            o_ref[...] = jnp.zeros_like(o_ref)

        rid = lax.broadcasted_iota(jnp.int32, (num_graphs, BN), 0)
        oh = (rid == b_ref[0]).astype(jnp.float32)
        rowok = (s * BN + lax.broadcasted_iota(jnp.int32, (BN, 1), 0)) < N
        hv = jnp.where(rowok, h_ref[...], 0.0)
        o_ref[...] += jnp.dot(oh, hv, preferred_element_type=jnp.float32, precision=jax.lax.Precision.HIGHEST)

    return pl.pallas_call(
        body,
        grid=(g,),
        in_specs=[
            pl.BlockSpec((1, 1, BN), lambda i: (i, 0, 0)),
            pl.BlockSpec((BN, F), lambda i: (i, 0)),
        ],
        out_specs=pl.BlockSpec((num_graphs, F), lambda i: (0, 0)),
        out_shape=jax.ShapeDtypeStruct((num_graphs, F), jnp.float32),
    )(b3, h)


def _mean_pool(h, batch, num_graphs):
    s = _sum_pool(h, batch, num_graphs)
    cnt = jax.ops.segment_sum(
        jnp.ones((h.shape[0],), jnp.float32), batch, num_segments=num_graphs + 1)
    return s / jnp.maximum(cnt[:num_graphs], 1.0)[:, None]


def _pad_heads(W, f, fp):
    shp = W.shape[:-1] + (HEADS, f)
    Wr = W.reshape(shp)
    pad = [(0, 0)] * (Wr.ndim - 1) + [(0, fp - f)]
    return jnp.pad(Wr, pad).reshape(W.shape[:-1] + (HEADS * fp,))


def _pad_cols(W, fp):
    return jnp.pad(W, [(0, 0)] * (W.ndim - 1) + [(0, fp - W.shape[-1])])


def _pad_rows(W, rp):
    return jnp.pad(W, [(0, rp - W.shape[0])] + [(0, 0)] * (W.ndim - 1))


def _build_pool_plan(score, ei, batch, emask, num_graphs, ratio=0.5):
    N = score.shape[0]
    order = jnp.lexsort((-score, batch))
    batch_sorted = batch[order]
    counts = jax.ops.segment_sum(jnp.ones_like(batch), batch, num_segments=num_graphs + 1)
    k = jnp.ceil(ratio * counts).astype(counts.dtype)
    starts = jnp.concatenate([jnp.zeros((1,), counts.dtype), jnp.cumsum(counts)[:-1]])
    rank = jnp.arange(N, dtype=counts.dtype) - starts[batch_sorted]
    keep_sorted = (rank < k[batch_sorted]) & (batch_sorted < num_graphs)
    order2 = jnp.argsort(~keep_sorted)
    perm = order[order2]
    nkeep = jnp.sum(keep_sorted.astype(counts.dtype))
    alive_new = jnp.arange(N, dtype=counts.dtype) < nkeep
    new_batch = jnp.where(alive_new, batch[perm], num_graphs)
    remap = jnp.zeros((N,), ei.dtype).at[perm].set(jnp.arange(N, dtype=ei.dtype))
    s0 = remap[ei[0]]
    d0 = remap[ei[1]]
    new_emask = emask & (s0 < nkeep) & (d0 < nkeep)
    new_ei = jnp.where(new_emask[None, :], jnp.stack([s0, d0]), 0)
    return perm, new_ei, new_emask, new_batch


def kernel(x, edge_index, edge_attr, batch, params):
    num_graphs = NUM_GRAPHS
    N = x.shape[0]
    h = _dense(x, params['node_W'], params['node_b'])
    ea = _dense(edge_attr, _pad_cols(params['edge_W'], 128),
                _pad_cols(params['edge_b'], 128))
    ei = edge_index.astype(jnp.int32)
    bt = batch.astype(jnp.int32)
    emask = jnp.ones((ei.shape[1],), bool)
    graph_out = 0.0
    for i in range(3):
        p = params['gat%d' % i]
        f, fp = HD[i + 1], HDP[i + 1]
        fprev = h.shape[1]
        src, dst = ei[0], ei[1]
        order = jnp.argsort(dst).astype(jnp.int32)
        src_s = src[order]
        dst_s = dst[order]
        em_s = emask[order]

        xl = _dense(h, _pad_rows(_pad_heads(p['Wl'], f, fp), fprev),
                    _pad_heads(p['bl'], f, fp), silu=False)
        xr = _dense(h, _pad_rows(_pad_heads(p['Wr'], f, fp), fprev),
                    _pad_heads(p['br'], f, fp), silu=False)
        ea_s = sc_gather(ea, order)
        ep = _dense(ea_s, _pad_rows(_pad_heads(p['We'], f, fp), 128),
                    None, silu=False)
        xls = sc_gather(xl, src_s)
        xrs = sc_gather(xr, dst_s)
        att_flat = _pad_cols(p['att'], fp).reshape(-1)
        logits = _edge_logits(xls, xrs, ep, att_flat, fp)

        mlog = jnp.where(em_s[:, None], logits, -jnp.inf)
        lmax = jax.ops.segment_max(mlog, dst_s, num_segments=N)
        lmax = jnp.where(jnp.isfinite(lmax), lmax, 0.0)
        ex = jnp.where(em_s[:, None], jnp.exp(logits - lmax[dst_s]), 0.0)
        den = jax.ops.segment_sum(ex, dst_s, num_segments=N)
        alpha = ex / (den[dst_s] + 1e-16)
        wT = jnp.zeros((8, ei.shape[1]), jnp.float32).at[:HEADS].set(alpha.T)

        tmp = (xls.reshape(-1, HEADS, fp) * alpha[:, :, None]).mean(axis=1)
        h = jax.ops.segment_sum(tmp, dst_s, num_segments=N) + _pad_cols(p['bias'], fp)

        pooled = _mean_pool(h, bt, num_graphs)[:, :f]
        graph_out = graph_out + _dense(
            pooled, params['gm%d_W' % i], params['gm%d_b' % i],
            params['gm%d_g' % i], params['gm%d_be' % i])

        if i < 2:
            q = params['pool%d' % i]
            h = jnp.where(h >= 0, h, 0.01 * h)
            hs = sc_gather(h, src_s)
            agg = jax.ops.segment_sum(
                jnp.where(em_s[:, None], hs, 0.0), dst_s, num_segments=N)
            sc_a = _dense(agg, _pad_cols(_pad_rows(q['W_rel'], fp), 128),
                          _pad_cols(q['b_rel'], 128), silu=False)
            sc_b = _dense(h, _pad_cols(_pad_rows(q['W_root'], fp), 128),
                          None, silu=False)
            score = jnp.tanh(sc_a[:, 0] + sc_b[:, 0])
            perm, new_ei, new_emask, new_bt = _build_pool_plan(
                score, ei, bt, emask, num_graphs)
            h = sc_gather(h, perm.astype(jnp.int32)) * score[perm][:, None]
            ei = new_ei
            emask = new_emask
            bt = new_bt

    return _dense(graph_out, params['fc_W'], params['fc_b'])
